# baseline (speedup 1.0000x reference)
import jax
import jax.numpy as jnp
from jax import lax
from jax.experimental import pallas as pl
from jax.experimental.pallas import tpu as pltpu
from jax.experimental.pallas import tpu_sc as plsc

B, H, W, C = 4, 224, 224, 96
P = H * W
NSEG = 64
ROWS = B * P
NC, NS = 2, 16
NW = NC * NS
RPW = ROWS // NW
NB = 224
NBLK = RPW // NB
GRP = NB // 16
CCH = C // 16
CPK = CCH // 2
AROWS = 72
SW = 128
CW = 128


def _body(img, labs, out, buf0, buf1, lab0, lab1, accs, accc, stgs,
          stgc, res8, shs, shc, sem0, sem1):
    cid = lax.axis_index("c")
    sid = lax.axis_index("s")
    wid = cid * NS + sid
    blocal = sid // 8
    row0 = wid * RPW

    bufs, labv, sems = (buf0, buf1), (lab0, lab1), (sem0, sem1)

    zf16 = jnp.zeros((16,), jnp.float32)
    zi16 = jnp.zeros((16,), jnp.int32)

    scope = jax.named_scope
    with scope("zero_acc"):
        def _zs(i, _):
            accs[i // (SW // 16), pl.ds((i % (SW // 16)) * 16, 16)] = zf16
            return 0
        lax.fori_loop(0, AROWS * (SW // 16), _zs, 0)

        def _zc(i, _):
            accc[i // (CW // 16), pl.ds((i % (CW // 16)) * 16, 16)] = zi16
            return 0
        lax.fori_loop(0, AROWS * (CW // 16), _zc, 0)

    def _start(g, ph):
        base = row0 + g * NB
        pltpu.async_copy(img.at[pl.ds(base, NB)], bufs[ph], sems[ph])
        pltpu.async_copy(labs.at[pl.ds(base, NB)], labv[ph], sems[ph])

    def _wait(g, ph):
        base = row0 + g * NB
        pltpu.make_async_copy(img.at[pl.ds(base, NB)], bufs[ph],
                              sems[ph]).wait()
        pltpu.make_async_copy(labs.at[pl.ds(base, NB)], labv[ph],
                              sems[ph]).wait()

    def _compute(bufp, labp):
        def _grp(g16, _):
            lv = labp[pl.ds(g16 * 16, 16)]
            rv = jnp.where(lv == 0, AROWS - 1, lv - 1)
            rows = [rv[k] for k in range(16)]
            for k in range(16):
                i = g16 * 16 + k
                xs = [bufp[i, pl.ds(j * 16, 16)] for j in range(CCH)]
                nzi = [jnp.where(x == 0.0, 0, 1) for x in xs]
                for j in range(CCH):
                    plsc.addupdate(accs.at[rows[k], pl.ds(j * 16, 16)], xs[j])
                for j in range(CPK):
                    pk = nzi[2 * j] + (nzi[2 * j + 1] << 16)
                    plsc.addupdate(accc.at[rows[k], pl.ds(j * 16, 16)], pk)
            return 0
        lax.fori_loop(0, GRP, _grp, 0)

    with scope("mainloop"):
        _start(0, 0)
        _start(1, 1)

        def _pair(p, _):
            for ph in range(2):
                g = p * 2 + ph
                with scope("wait"):
                    _wait(g, ph)
                with scope("compute"):
                    _compute(bufs[ph], labv[ph])

                @pl.when(g + 2 < NBLK)
                def _():
                    _start(g + 2, ph)
            return 0
        lax.fori_loop(0, NBLK // 2, _pair, 0)

    scope2 = jax.named_scope("combine")
    scope2.__enter__()
    pltpu.sync_copy(accs, shs.at[pl.ds(sid * AROWS, AROWS)])
    pltpu.sync_copy(accc, shc.at[pl.ds(sid * AROWS, AROWS)])
    plsc.subcore_barrier()

    s0 = (sid % 8) * 8
    for k in range(8):
        srow = (blocal * 8 + k) * AROWS + s0
        pltpu.sync_copy(shs.at[pl.ds(srow, 8)], stgs.at[pl.ds(k * 8, 8)])
        pltpu.sync_copy(shc.at[pl.ds(srow, 8)], stgc.at[pl.ds(k * 8, 8)])

    for r in range(8):
        for j in range(CPK):
            cp = stgc[r, pl.ds(j * 16, 16)]
            s_a = stgs[r, pl.ds((2 * j) * 16, 16)]
            s_b = stgs[r, pl.ds((2 * j + 1) * 16, 16)]
            ca = cp & 0xFFFF
            cb = cp >> 16
            for k in range(1, 8):
                cp = stgc[k * 8 + r, pl.ds(j * 16, 16)]
                s_a = s_a + stgs[k * 8 + r, pl.ds((2 * j) * 16, 16)]
                s_b = s_b + stgs[k * 8 + r, pl.ds((2 * j + 1) * 16, 16)]
                ca = ca + (cp & 0xFFFF)
                cb = cb + (cp >> 16)
            res8[r, pl.ds((2 * j) * 16, 16)] = s_a / ca.astype(jnp.float32)
            res8[r, pl.ds((2 * j + 1) * 16, 16)] = s_b / cb.astype(jnp.float32)
    outrow0 = (cid * 2 + blocal) * NSEG + s0
    pltpu.sync_copy(res8, out.at[pl.ds(outrow0, 8)])
    scope2.__exit__(None, None, None)


@jax.jit
def _confidence_sc(img, labs):
    mesh = plsc.VectorSubcoreMesh(core_axis_name="c", subcore_axis_name="s")
    return pl.kernel(
        _body,
        out_type=jax.ShapeDtypeStruct((B * NSEG, C), jnp.float32),
        mesh=mesh,
        scratch_types=[
            pltpu.VMEM((NB, C), jnp.float32),
            pltpu.VMEM((NB, C), jnp.float32),
            pltpu.VMEM((NB,), jnp.int32),
            pltpu.VMEM((NB,), jnp.int32),
            pltpu.VMEM((AROWS, SW), jnp.float32),
            pltpu.VMEM((AROWS, CW), jnp.int32),
            pltpu.VMEM((64, SW), jnp.float32),
            pltpu.VMEM((64, CW), jnp.int32),
            pltpu.VMEM((8, C), jnp.float32),
            pltpu.VMEM_SHARED((NS * AROWS, SW), jnp.float32),
            pltpu.VMEM_SHARED((NS * AROWS, CW), jnp.int32),
            pltpu.SemaphoreType.DMA,
            pltpu.SemaphoreType.DMA,
        ],
        name="slic_confidence_sc",
    )(img, labs)


def kernel(image, slic):
    img = image.reshape(ROWS, C)
    labs = slic.reshape(ROWS)
    out = _confidence_sc(img, labs)
    return out.reshape(B, NSEG, C)


# speedup vs baseline: 2.3583x; 2.3583x over previous
import jax
import jax.numpy as jnp
from jax import lax
from jax.experimental import pallas as pl
from jax.experimental.pallas import tpu as pltpu
from jax.experimental.pallas import tpu_sc as plsc

B, H, W, C = 4, 224, 224, 96
P = H * W
NSEG = 64
ROWS = B * P
NC, NS = 2, 16
NW = NC * NS
RPW = ROWS // NW
NB = 224
NBLK = RPW // NB
GRP = NB // 16
CCH = C // 16
CPK = CCH // 2
AROWS = 72
SW = 128
CW = 128


def _body(img, labs, out, buf0, buf1, lab0, lab1, accs, accc, stgs,
          stgc, res8, shs, shc, sem0, sem1):
    cid = lax.axis_index("c")
    sid = lax.axis_index("s")
    wid = cid * NS + sid
    blocal = sid // 8
    bidx = wid // (NW // B)
    h0 = (wid % (NW // B)) * NBLK

    bufs, labv, sems = (buf0, buf1), (lab0, lab1), (sem0, sem1)

    zf16 = jnp.zeros((16,), jnp.float32)
    zi16 = jnp.zeros((16,), jnp.int32)

    scope = jax.named_scope
    with scope("zero_acc"):
        def _zs(i, _):
            accs[i // (SW // 16), pl.ds((i % (SW // 16)) * 16, 16)] = zf16
            return 0
        lax.fori_loop(0, AROWS * (SW // 16), _zs, 0)

        def _zc(i, _):
            accc[i // (CW // 16), pl.ds((i % (CW // 16)) * 16, 16)] = zi16
            return 0
        lax.fori_loop(0, AROWS * (CW // 16), _zc, 0)

    def _start(g, ph):
        base = (bidx * (NW // B) + (wid % (NW // B))) * RPW + g * NB
        pltpu.async_copy(img.at[bidx, h0 + g], bufs[ph], sems[ph])
        pltpu.async_copy(labs.at[pl.ds(base, NB)], labv[ph], sems[ph])

    def _wait(g, ph):
        base = (bidx * (NW // B) + (wid % (NW // B))) * RPW + g * NB
        pltpu.make_async_copy(img.at[bidx, h0 + g], bufs[ph],
                              sems[ph]).wait()
        pltpu.make_async_copy(labs.at[pl.ds(base, NB)], labv[ph],
                              sems[ph]).wait()

    def _compute(bufp, labp):
        def _grp(g16, _):
            lv = labp[pl.ds(g16 * 16, 16)]
            rv = jnp.where(lv == 0, AROWS - 1, lv - 1)
            rows = [rv[k] for k in range(16)]
            for k in range(16):
                i = g16 * 16 + k
                xs = [bufp[i, pl.ds(j * 16, 16)] for j in range(CCH)]
                nzi = [jnp.where(x == 0.0, 0, 1) for x in xs]
                for j in range(CCH):
                    plsc.addupdate(accs.at[rows[k], pl.ds(j * 16, 16)], xs[j])
                for j in range(CPK):
                    pk = nzi[2 * j] + (nzi[2 * j + 1] << 16)
                    plsc.addupdate(accc.at[rows[k], pl.ds(j * 16, 16)], pk)
            return 0
        lax.fori_loop(0, GRP, _grp, 0)

    with scope("mainloop"):
        _start(0, 0)
        _start(1, 1)

        def _pair(p, _):
            for ph in range(2):
                g = p * 2 + ph
                with scope("wait"):
                    _wait(g, ph)
                with scope("compute"):
                    _compute(bufs[ph], labv[ph])

                @pl.when(g + 2 < NBLK)
                def _():
                    _start(g + 2, ph)
            return 0
        lax.fori_loop(0, NBLK // 2, _pair, 0)

    scope2 = jax.named_scope("combine")
    scope2.__enter__()
    pltpu.sync_copy(accs, shs.at[pl.ds(sid * AROWS, AROWS)])
    pltpu.sync_copy(accc, shc.at[pl.ds(sid * AROWS, AROWS)])
    plsc.subcore_barrier()

    s0 = (sid % 8) * 8
    for k in range(8):
        srow = (blocal * 8 + k) * AROWS + s0
        pltpu.sync_copy(shs.at[pl.ds(srow, 8)], stgs.at[pl.ds(k * 8, 8)])
        pltpu.sync_copy(shc.at[pl.ds(srow, 8)], stgc.at[pl.ds(k * 8, 8)])

    for r in range(8):
        for j in range(CPK):
            cp = stgc[r, pl.ds(j * 16, 16)]
            s_a = stgs[r, pl.ds((2 * j) * 16, 16)]
            s_b = stgs[r, pl.ds((2 * j + 1) * 16, 16)]
            ca = cp & 0xFFFF
            cb = cp >> 16
            for k in range(1, 8):
                cp = stgc[k * 8 + r, pl.ds(j * 16, 16)]
                s_a = s_a + stgs[k * 8 + r, pl.ds((2 * j) * 16, 16)]
                s_b = s_b + stgs[k * 8 + r, pl.ds((2 * j + 1) * 16, 16)]
                ca = ca + (cp & 0xFFFF)
                cb = cb + (cp >> 16)
            res8[r, pl.ds((2 * j) * 16, 16)] = s_a / ca.astype(jnp.float32)
            res8[r, pl.ds((2 * j + 1) * 16, 16)] = s_b / cb.astype(jnp.float32)
    outrow0 = (cid * 2 + blocal) * NSEG + s0
    pltpu.sync_copy(res8, out.at[pl.ds(outrow0, 8)])
    scope2.__exit__(None, None, None)


@jax.jit
def _confidence_sc(img, labs):
    mesh = plsc.VectorSubcoreMesh(core_axis_name="c", subcore_axis_name="s")
    return pl.kernel(
        _body,
        out_type=jax.ShapeDtypeStruct((B * NSEG, C), jnp.float32),
        mesh=mesh,
        scratch_types=[
            pltpu.VMEM((NB, C), jnp.float32),
            pltpu.VMEM((NB, C), jnp.float32),
            pltpu.VMEM((NB,), jnp.int32),
            pltpu.VMEM((NB,), jnp.int32),
            pltpu.VMEM((AROWS, SW), jnp.float32),
            pltpu.VMEM((AROWS, CW), jnp.int32),
            pltpu.VMEM((64, SW), jnp.float32),
            pltpu.VMEM((64, CW), jnp.int32),
            pltpu.VMEM((8, C), jnp.float32),
            pltpu.VMEM_SHARED((NS * AROWS, SW), jnp.float32),
            pltpu.VMEM_SHARED((NS * AROWS, CW), jnp.int32),
            pltpu.SemaphoreType.DMA,
            pltpu.SemaphoreType.DMA,
        ],
        name="slic_confidence_sc",
    )(img, labs)


def kernel(image, slic):
    labs = slic.reshape(ROWS)
    out = _confidence_sc(image, labs)
    return out.reshape(B, NSEG, C)
